# baseline (speedup 1.0000x reference)
import jax
import jax.numpy as jnp
from jax.experimental import pallas as pl
from jax.experimental.pallas import tpu as pltpu

ALPHA = 0.1
ROW_BLOCK = 256


def _gconv_block(a_ref, x_ref, x0_ref, mk_ref, bt_ref, out_ref, xt):
    i = pl.program_id(0)
    B = x_ref.shape[0]
    D = x_ref.shape[2]

    @pl.when(i == 0)
    def _():
        N = x_ref.shape[1]
        step = 256
        for k in range(N // step):
            sl = pl.ds(k * step, step)
            xt[sl, :] = pltpu.einshape(
                "bnd->n(bd)", x_ref[:, sl, :].astype(jnp.bfloat16))

    agg = jnp.dot(a_ref[...].astype(jnp.bfloat16), xt[...],
                  preferred_element_type=jnp.float32)

    x0t = pltpu.einshape("bnd->n(bd)", x0_ref[...])
    h = (1.0 - ALPHA) * agg + ALPHA * x0t
    hw = jnp.dot(h.astype(jnp.bfloat16), mk_ref[...],
                 preferred_element_type=jnp.float32)
    o = jax.nn.gelu(hw + bt_ref[...])
    out_ref[...] = pltpu.einshape("n(bd)->bnd", o, b=B, d=D)


def kernel(x, x0, adj, W, b):
    B, N, D = x.shape
    BD = B * D
    m = 0.5 * (jnp.eye(D, dtype=jnp.float32) + W)
    mk = jnp.kron(jnp.eye(B, dtype=jnp.float32), m).astype(jnp.bfloat16)
    bt = jnp.tile(b, B).reshape(1, BD)

    grid = (N // ROW_BLOCK,)
    out = pl.pallas_call(
        _gconv_block,
        grid=grid,
        in_specs=[
            pl.BlockSpec((ROW_BLOCK, N), lambda i: (i, 0)),
            pl.BlockSpec((B, N, D), lambda i: (0, 0, 0)),
            pl.BlockSpec((B, ROW_BLOCK, D), lambda i: (0, i, 0)),
            pl.BlockSpec((BD, BD), lambda i: (0, 0)),
            pl.BlockSpec((1, BD), lambda i: (0, 0)),
        ],
        out_specs=pl.BlockSpec((B, ROW_BLOCK, D), lambda i: (0, i, 0)),
        out_shape=jax.ShapeDtypeStruct((B, N, D), jnp.float32),
        scratch_shapes=[
            pltpu.VMEM((N, BD), jnp.bfloat16),
        ],
        compiler_params=pltpu.CompilerParams(
            dimension_semantics=("arbitrary",),
            vmem_limit_bytes=63 * 1024 * 1024 + 512 * 1024,
        ),
    )(adj, x, x0, mk, bt)
    return out


# speedup vs baseline: 1.2999x; 1.2999x over previous
import jax
import jax.numpy as jnp
from jax.experimental import pallas as pl
from jax.experimental.pallas import tpu as pltpu

ALPHA = 0.1
ROW_BLOCK = 128


def _gconv_block(a_ref, xt_ref, x0t_ref, mk_ref, bt_ref, out_ref):
    agg = jnp.dot(a_ref[...], xt_ref[...], preferred_element_type=jnp.float32)
    h = (1.0 - ALPHA) * agg + ALPHA * x0t_ref[...]
    hw = jnp.dot(h, mk_ref[...], preferred_element_type=jnp.float32)
    out_ref[...] = jax.nn.gelu(hw + bt_ref[...])


def kernel(x, x0, adj, W, b):
    B, N, D = x.shape
    BD = B * D
    xt = jnp.transpose(x, (1, 0, 2)).reshape(N, BD)
    x0t = jnp.transpose(x0, (1, 0, 2)).reshape(N, BD)
    m = 0.5 * (jnp.eye(D, dtype=jnp.float32) + W)
    mk = jnp.kron(jnp.eye(B, dtype=jnp.float32), m)
    bt = jnp.tile(b, B).reshape(1, BD)

    grid = (N // ROW_BLOCK,)
    outt = pl.pallas_call(
        _gconv_block,
        grid=grid,
        in_specs=[
            pl.BlockSpec((ROW_BLOCK, N), lambda i: (i, 0)),
            pl.BlockSpec((N, BD), lambda i: (0, 0)),
            pl.BlockSpec((ROW_BLOCK, BD), lambda i: (i, 0)),
            pl.BlockSpec((BD, BD), lambda i: (0, 0)),
            pl.BlockSpec((1, BD), lambda i: (0, 0)),
        ],
        out_specs=pl.BlockSpec((ROW_BLOCK, BD), lambda i: (i, 0)),
        out_shape=jax.ShapeDtypeStruct((N, BD), jnp.float32),
        compiler_params=pltpu.CompilerParams(
            dimension_semantics=("parallel",),
            vmem_limit_bytes=63 * 1024 * 1024,
        ),
    )(adj, xt, x0t, mk, bt)
    return jnp.transpose(outt.reshape(N, B, D), (1, 0, 2))


# speedup vs baseline: 1.3842x; 1.0649x over previous
import jax
import jax.numpy as jnp
from jax.experimental import pallas as pl
from jax.experimental.pallas import tpu as pltpu

ALPHA = 0.1
ROW_BLOCK = 512


def _gconv_block(a_ref, xt_ref, x0t_ref, mk_ref, bt_ref, out_ref):
    a_bf = a_ref[...].astype(jnp.bfloat16)
    agg = jnp.dot(a_bf, xt_ref[...], preferred_element_type=jnp.float32)
    h = (1.0 - ALPHA) * agg + ALPHA * x0t_ref[...].astype(jnp.float32)
    hw = jnp.dot(h.astype(jnp.bfloat16), mk_ref[...],
                 preferred_element_type=jnp.float32)
    out_ref[...] = jax.nn.gelu(hw + bt_ref[...]).astype(jnp.bfloat16)


def kernel(x, x0, adj, W, b):
    B, N, D = x.shape
    BD = B * D
    xt = jnp.transpose(x, (1, 0, 2)).reshape(N, BD).astype(jnp.bfloat16)
    x0t = jnp.transpose(x0, (1, 0, 2)).reshape(N, BD).astype(jnp.bfloat16)
    m = 0.5 * (jnp.eye(D, dtype=jnp.float32) + W)
    mk = jnp.kron(jnp.eye(B, dtype=jnp.float32), m).astype(jnp.bfloat16)
    bt = jnp.tile(b, B).reshape(1, BD)

    grid = (N // ROW_BLOCK,)
    outt = pl.pallas_call(
        _gconv_block,
        grid=grid,
        in_specs=[
            pl.BlockSpec((ROW_BLOCK, N), lambda i: (i, 0)),
            pl.BlockSpec((N, BD), lambda i: (0, 0)),
            pl.BlockSpec((ROW_BLOCK, BD), lambda i: (i, 0)),
            pl.BlockSpec((BD, BD), lambda i: (0, 0)),
            pl.BlockSpec((1, BD), lambda i: (0, 0)),
        ],
        out_specs=pl.BlockSpec((ROW_BLOCK, BD), lambda i: (i, 0)),
        out_shape=jax.ShapeDtypeStruct((N, BD), jnp.bfloat16),
        compiler_params=pltpu.CompilerParams(
            dimension_semantics=("parallel",),
            vmem_limit_bytes=63 * 1024 * 1024,
            allow_input_fusion=[True, True, True, True, True],
        ),
    )(adj, xt, x0t, mk, bt)
    return jnp.transpose(outt.reshape(N, B, D), (1, 0, 2)).astype(jnp.float32)


# speedup vs baseline: 1.4099x; 1.0186x over previous
import jax
import jax.numpy as jnp
from jax.experimental import pallas as pl
from jax.experimental.pallas import tpu as pltpu

ALPHA = 0.1
ROW_BLOCK = 512


def _gconv_block(a_ref, xt_ref, x0t_ref, mk_ref, bt_ref, out_ref):
    a_bf = a_ref[...].astype(jnp.bfloat16)
    agg = jnp.dot(a_bf, xt_ref[...], preferred_element_type=jnp.float32)
    h = (1.0 - ALPHA) * agg + ALPHA * x0t_ref[...]
    hw = jnp.dot(h.astype(jnp.bfloat16), mk_ref[...],
                 preferred_element_type=jnp.float32)
    out_ref[...] = jax.nn.gelu(hw + bt_ref[...])


def kernel(x, x0, adj, W, b):
    B, N, D = x.shape
    BD = B * D
    xt = jnp.transpose(x, (1, 0, 2)).reshape(N, BD).astype(jnp.bfloat16)
    x0t = jnp.transpose(x0, (1, 0, 2)).reshape(N, BD)
    m = 0.5 * (jnp.eye(D, dtype=jnp.float32) + W)
    mk = jnp.kron(jnp.eye(B, dtype=jnp.float32), m).astype(jnp.bfloat16)
    bt = jnp.tile(b, B).reshape(1, BD)

    grid = (N // ROW_BLOCK,)
    outt = pl.pallas_call(
        _gconv_block,
        grid=grid,
        in_specs=[
            pl.BlockSpec((ROW_BLOCK, N), lambda i: (i, 0)),
            pl.BlockSpec((N, BD), lambda i: (0, 0)),
            pl.BlockSpec((ROW_BLOCK, BD), lambda i: (i, 0)),
            pl.BlockSpec((BD, BD), lambda i: (0, 0)),
            pl.BlockSpec((1, BD), lambda i: (0, 0)),
        ],
        out_specs=pl.BlockSpec((ROW_BLOCK, BD), lambda i: (i, 0)),
        out_shape=jax.ShapeDtypeStruct((N, BD), jnp.float32),
        compiler_params=pltpu.CompilerParams(
            dimension_semantics=("parallel",),
        ),
    )(adj, xt, x0t, mk, bt)
    return jnp.transpose(outt.reshape(N, B, D), (1, 0, 2))


# speedup vs baseline: 1.5603x; 1.1067x over previous
import jax
import jax.numpy as jnp
from jax.experimental import pallas as pl
from jax.experimental.pallas import tpu as pltpu

ALPHA = 0.1
ROW_BLOCK = 512


def _gconv_block(a_ref, xtt_ref, x0t_ref, mk_ref, bt_ref, out_ref):
    a_bf = a_ref[...].astype(jnp.bfloat16)
    agg = jax.lax.dot_general(
        a_bf, xtt_ref[...], (((1,), (1,)), ((), ())),
        preferred_element_type=jnp.float32)
    h = (1.0 - ALPHA) * agg + ALPHA * x0t_ref[...]
    hw = jnp.dot(h.astype(jnp.bfloat16), mk_ref[...],
                 preferred_element_type=jnp.float32)
    out_ref[...] = jax.nn.gelu(hw + bt_ref[...])


def kernel(x, x0, adj, W, b):
    B, N, D = x.shape
    BD = B * D
    xtt = jnp.transpose(x, (0, 2, 1)).reshape(BD, N).astype(jnp.bfloat16)
    x0t = jnp.transpose(x0, (1, 0, 2)).reshape(N, BD)
    m = 0.5 * (jnp.eye(D, dtype=jnp.float32) + W)
    mk = jnp.kron(jnp.eye(B, dtype=jnp.float32), m).astype(jnp.bfloat16)
    bt = jnp.tile(b, B).reshape(1, BD)

    grid = (N // ROW_BLOCK,)
    outt = pl.pallas_call(
        _gconv_block,
        grid=grid,
        in_specs=[
            pl.BlockSpec((ROW_BLOCK, N), lambda i: (i, 0)),
            pl.BlockSpec((BD, N), lambda i: (0, 0)),
            pl.BlockSpec((ROW_BLOCK, BD), lambda i: (i, 0)),
            pl.BlockSpec((BD, BD), lambda i: (0, 0)),
            pl.BlockSpec((1, BD), lambda i: (0, 0)),
        ],
        out_specs=pl.BlockSpec((ROW_BLOCK, BD), lambda i: (i, 0)),
        out_shape=jax.ShapeDtypeStruct((N, BD), jnp.float32),
        compiler_params=pltpu.CompilerParams(
            dimension_semantics=("parallel",),
        ),
    )(adj, xtt, x0t, mk, bt)
    return jnp.transpose(outt.reshape(N, B, D), (1, 0, 2))
